# baseline (speedup 1.0000x reference)
import functools

import jax
import jax.numpy as jnp
from jax import lax
from jax.experimental import pallas as pl
from jax.experimental.pallas import tpu as pltpu
from jax.experimental.pallas import tpu_sc as plsc

SEQ = 8192
DIM = 2048
TOT = 4 * 8192
NC, NS = 2, 16
GW = 128
NGRP = DIM // GW // NC
POS_W = TOT // NS
APOS = 1792
CPOS = 64
NCH = APOS // CPOS
BPOS = POS_W - APOS
BCH = 8
NBG = 2

_mesh = plsc.VectorSubcoreMesh(core_axis_name="c", subcore_axis_name="s")


@functools.partial(
    pl.kernel,
    out_type=jax.ShapeDtypeStruct((TOT, DIM), jnp.float32),
    mesh=_mesh,
    scratch_types=[
        pltpu.VMEM((POS_W + 16,), jnp.int32),
        pltpu.VMEM_SHARED((SEQ, GW), jnp.float32),
        [pltpu.VMEM((CPOS, GW), jnp.float32)] * 3,
        pltpu.VMEM_SHARED((NS * 2 * BCH, DIM), jnp.float32),
        pltpu.SemaphoreType.DMA,
        [pltpu.SemaphoreType.DMA] * 3,
        [pltpu.SemaphoreType.DMA] * 3,
        [pltpu.SemaphoreType.DMA] * 2,
        [pltpu.SemaphoreType.DMA] * 2,
    ],
)
def _gather_sc(ids_hbm, table_hbm, out_hbm, idx_v, slab, tbufs, bslab,
               ssem, gsems, psems, bgs, bps):
    cid = lax.axis_index("c")
    sid = lax.axis_index("s")
    pos0 = sid * POS_W

    pltpu.sync_copy(ids_hbm.at[sid], idx_v.at[pl.ds(0, POS_W)])

    def col0(g):
        return (cid * NGRP + g) * GW

    def stage(g):
        src = table_hbm.at[pl.ds(sid * (SEQ // NS), SEQ // NS), pl.ds(col0(g), GW)]
        pltpu.async_copy(src, slab.at[pl.ds(sid * (SEQ // NS), SEQ // NS)], ssem)

    def swait():
        src = table_hbm.at[pl.ds(0, SEQ // NS), pl.ds(0, GW)]
        pltpu.make_async_copy(src, slab.at[pl.ds(0, SEQ // NS)], ssem).wait()

    def gather(ch, tb):
        idx = idx_v.at[pl.ds(ch * CPOS, CPOS)]
        pltpu.async_copy(slab.at[idx], tbufs[tb], gsems[tb])

    def gwait(tb):
        idx = idx_v.at[pl.ds(0, CPOS)]
        pltpu.make_async_copy(slab.at[idx], tbufs[tb], gsems[tb]).wait()

    def put(g, ch, tb):
        dst = out_hbm.at[pl.ds(pos0 + ch * CPOS, CPOS), pl.ds(col0(g), GW)]
        pltpu.async_copy(tbufs[tb], dst, psems[tb])

    def pwait(tb):
        dst = out_hbm.at[pl.ds(0, CPOS), pl.ds(0, GW)]
        pltpu.make_async_copy(tbufs[tb], dst, psems[tb]).wait()

    boff = APOS + cid * (BPOS // NC)

    def bsl(bb):
        return bslab.at[pl.ds((sid * 2 + bb) * BCH, BCH)]

    def bgather(q, bb):
        vec = idx_v[pl.ds(boff + q * BCH, 16)]
        sl = bsl(bb)
        for k in range(BCH):
            pltpu.async_copy(
                table_hbm.at[pl.ds(vec[k], 1)], sl.at[pl.ds(k, 1)], bgs[bb]
            )

    def bgwait(bb):
        sl = bsl(bb)
        for k in range(BCH):
            pltpu.make_async_copy(
                table_hbm.at[pl.ds(0, 1)], sl.at[pl.ds(k, 1)], bgs[bb]
            ).wait()

    def bput(q, bb):
        dst = out_hbm.at[pl.ds(pos0 + boff + q * BCH, BCH)]
        pltpu.async_copy(bsl(bb), dst, bps[bb])

    def bpwait(bb):
        dst = out_hbm.at[pl.ds(pos0 + boff, BCH)]
        pltpu.make_async_copy(bsl(bb), dst, bps[bb]).wait()

    def process(g):
        gather(0, 0)
        gather(1, 1)
        gather(2, 2)
        for ch in range(NCH):
            tb = ch % 3
            gwait(tb)
            put(g, ch, tb)
            if ch == NCH - 4:
                bgather(NBG * g, 0)
            if ch == NCH - 2:
                bgather(NBG * g + 1, 1)
            if ch + 3 < NCH:
                pwait(tb)
                gather(ch + 3, tb)

    stage(0)

    def body(g, carry):
        swait()
        plsc.subcore_barrier()
        process(g)
        plsc.subcore_barrier()

        @pl.when(g < NGRP - 1)
        def _():
            stage(g + 1)
        bgwait(0)
        bput(NBG * g, 0)
        bgwait(1)
        bput(NBG * g + 1, 1)
        for tb in range(3):
            pwait(tb)
        bpwait(0)
        bpwait(1)
        return carry

    lax.fori_loop(0, NGRP, body, 0)


def kernel(position_ids, table):
    ids = position_ids.reshape(NS, POS_W).astype(jnp.int32)
    out = _gather_sc(ids, table)
    return out.reshape(position_ids.shape[0], position_ids.shape[1], DIM)


# speedup vs baseline: 1.0460x; 1.0460x over previous
import functools

import jax
import jax.numpy as jnp
from jax import lax
from jax.experimental import pallas as pl
from jax.experimental.pallas import tpu as pltpu
from jax.experimental.pallas import tpu_sc as plsc

SEQ = 8192
DIM = 2048
TOT = 4 * 8192
NC, NS = 2, 16
GW = 128
NGRP = DIM // GW // NC
POS_W = TOT // NS
CPOS = 128
NCH = POS_W // CPOS

_mesh = plsc.VectorSubcoreMesh(core_axis_name="c", subcore_axis_name="s")


@functools.partial(
    pl.kernel,
    out_type=jax.ShapeDtypeStruct((TOT, DIM), jnp.float32),
    mesh=_mesh,
    scratch_types=[
        pltpu.VMEM((POS_W,), jnp.int32),
        pltpu.VMEM_SHARED((SEQ, GW), jnp.float32),
        [pltpu.VMEM((CPOS, GW), jnp.float32)] * 3,
        pltpu.SemaphoreType.DMA,
        [pltpu.SemaphoreType.DMA] * 3,
        [pltpu.SemaphoreType.DMA] * 3,
    ],
)
def _gather_sc(ids_hbm, table_hbm, out_hbm, idx_v, slab, tbufs, ssem, gsems, psems):
    cid = lax.axis_index("c")
    sid = lax.axis_index("s")
    pos0 = sid * POS_W

    pltpu.sync_copy(ids_hbm.at[sid], idx_v)

    def col0(g):
        return (cid * NGRP + g) * GW

    SPC = SEQ // NS // 4

    def stage(g):
        for r in range(4):
            r0 = sid * (SEQ // NS) + r * SPC
            src = table_hbm.at[pl.ds(r0, SPC), pl.ds(col0(g), GW)]
            pltpu.async_copy(src, slab.at[pl.ds(r0, SPC)], ssem)

    def swait():
        for _ in range(4):
            src = table_hbm.at[pl.ds(0, SPC), pl.ds(0, GW)]
            pltpu.make_async_copy(src, slab.at[pl.ds(0, SPC)], ssem).wait()

    def gather(ch, tb):
        idx = idx_v.at[pl.ds(ch * CPOS, CPOS)]
        pltpu.async_copy(slab.at[idx], tbufs[tb], gsems[tb])

    def gwait(tb):
        idx = idx_v.at[pl.ds(0, CPOS)]
        pltpu.make_async_copy(slab.at[idx], tbufs[tb], gsems[tb]).wait()

    def put(g, ch, tb):
        dst = out_hbm.at[pl.ds(pos0 + ch * CPOS, CPOS), pl.ds(col0(g), GW)]
        pltpu.async_copy(tbufs[tb], dst, psems[tb])

    def pwait(tb):
        dst = out_hbm.at[pl.ds(0, CPOS), pl.ds(0, GW)]
        pltpu.make_async_copy(tbufs[tb], dst, psems[tb]).wait()

    def process(g):
        gather(0, 0)
        gather(1, 1)
        gather(2, 2)
        for ch in range(NCH):
            tb = ch % 3
            gwait(tb)
            put(g, ch, tb)
            if ch + 3 < NCH:
                pwait(tb)
                gather(ch + 3, tb)

    stage(0)

    def body(g, carry):
        swait()
        plsc.subcore_barrier()
        process(g)
        plsc.subcore_barrier()

        @pl.when(g < NGRP - 1)
        def _():
            stage(g + 1)
        for tb in range(3):
            pwait(tb)
        return carry

    lax.fori_loop(0, NGRP, body, 0)


def kernel(position_ids, table):
    ids = position_ids.reshape(NS, POS_W).astype(jnp.int32)
    out = _gather_sc(ids, table)
    return out.reshape(position_ids.shape[0], position_ids.shape[1], DIM)


# speedup vs baseline: 1.0572x; 1.0107x over previous
import functools

import jax
import jax.numpy as jnp
from jax import lax
from jax.experimental import pallas as pl
from jax.experimental.pallas import tpu as pltpu
from jax.experimental.pallas import tpu_sc as plsc

SEQ = 8192
DIM = 2048
TOT = 4 * 8192
NC, NS = 2, 16
GW = 128
NGRP = DIM // GW // NC
POS_W = TOT // NS
CPOS = 64
NCH = POS_W // CPOS
HALF = SEQ // 2
SPC = HALF // NS

_mesh = plsc.VectorSubcoreMesh(core_axis_name="c", subcore_axis_name="s")


@functools.partial(
    pl.kernel,
    out_type=jax.ShapeDtypeStruct((TOT, DIM), jnp.float32),
    mesh=_mesh,
    scratch_types=[
        pltpu.VMEM((POS_W,), jnp.int32),
        pltpu.VMEM_SHARED((SEQ, GW), jnp.float32),
        pltpu.VMEM_SHARED((HALF, GW), jnp.float32),
        [pltpu.VMEM((CPOS, GW), jnp.float32)] * 3,
        pltpu.SemaphoreType.DMA,
        pltpu.SemaphoreType.DMA,
        [pltpu.SemaphoreType.DMA] * 3,
        [pltpu.SemaphoreType.DMA] * 3,
    ],
)
def _gather_sc(ids_hbm, table_hbm, out_hbm, idx_v, slab, half, tbufs,
               ssem, hsem, gsems, psems):
    cid = lax.axis_index("c")
    sid = lax.axis_index("s")
    pos0 = sid * POS_W

    pltpu.sync_copy(ids_hbm.at[sid], idx_v)

    def col0(g):
        return (cid * NGRP + g) * GW

    def stage_lower(g):
        r0 = sid * SPC
        src = table_hbm.at[pl.ds(r0, SPC), pl.ds(col0(g), GW)]
        pltpu.async_copy(src, slab.at[pl.ds(r0, SPC)], ssem)

    def swait():
        src = table_hbm.at[pl.ds(0, SPC), pl.ds(0, GW)]
        pltpu.make_async_copy(src, slab.at[pl.ds(0, SPC)], ssem).wait()

    def prefetch(g):
        r0 = sid * SPC
        src = table_hbm.at[pl.ds(HALF + r0, SPC), pl.ds(col0(g), GW)]
        pltpu.async_copy(src, half.at[pl.ds(r0, SPC)], hsem)

    def hwait():
        src = table_hbm.at[pl.ds(0, SPC), pl.ds(0, GW)]
        pltpu.make_async_copy(src, half.at[pl.ds(0, SPC)], hsem).wait()

    def relay():
        for r in range(SPC // CPOS):
            r0 = sid * SPC + r * CPOS
            pltpu.sync_copy(half.at[pl.ds(r0, CPOS)], tbufs[0])
            pltpu.sync_copy(tbufs[0], slab.at[pl.ds(HALF + r0, CPOS)])

    def gather(ch, tb):
        idx = idx_v.at[pl.ds(ch * CPOS, CPOS)]
        pltpu.async_copy(slab.at[idx], tbufs[tb], gsems[tb])

    def gwait(tb):
        idx = idx_v.at[pl.ds(0, CPOS)]
        pltpu.make_async_copy(slab.at[idx], tbufs[tb], gsems[tb]).wait()

    def put(g, ch, tb):
        dst = out_hbm.at[pl.ds(pos0 + ch * CPOS, CPOS), pl.ds(col0(g), GW)]
        pltpu.async_copy(tbufs[tb], dst, psems[tb])

    def pwait(tb):
        dst = out_hbm.at[pl.ds(0, CPOS), pl.ds(0, GW)]
        pltpu.make_async_copy(tbufs[tb], dst, psems[tb]).wait()

    def process(g):
        gather(0, 0)
        gather(1, 1)
        gather(2, 2)
        for ch in range(NCH):
            tb = ch % 3
            gwait(tb)
            put(g, ch, tb)
            if ch == 2:
                @pl.when(g < NGRP - 1)
                def _():
                    prefetch(g + 1)
            if ch + 3 < NCH:
                pwait(tb)
                gather(ch + 3, tb)

    stage_lower(0)
    prefetch(0)
    hwait()
    relay()

    def body(g, carry):
        swait()
        plsc.subcore_barrier()
        process(g)
        plsc.subcore_barrier()

        @pl.when(g < NGRP - 1)
        def _():
            stage_lower(g + 1)

        pwait(0)

        @pl.when(g < NGRP - 1)
        def _():
            hwait()
            relay()

        pwait(1)
        pwait(2)
        return carry

    lax.fori_loop(0, NGRP, body, 0)


def kernel(position_ids, table):
    ids = position_ids.reshape(NS, POS_W).astype(jnp.int32)
    out = _gather_sc(ids, table)
    return out.reshape(position_ids.shape[0], position_ids.shape[1], DIM)


# speedup vs baseline: 1.1185x; 1.0580x over previous
import functools

import jax
import jax.numpy as jnp
from jax import lax
from jax.experimental import pallas as pl
from jax.experimental.pallas import tpu as pltpu
from jax.experimental.pallas import tpu_sc as plsc

SEQ = 8192
DIM = 2048
TOT = 4 * 8192
NC, NS = 2, 16
GW = 128
NGRP = DIM // GW // NC
POS_W = TOT // NS
CPOS = 64
NCH = POS_W // CPOS
HALF = SEQ // 2
SPC = HALF // NS

_mesh = plsc.VectorSubcoreMesh(core_axis_name="c", subcore_axis_name="s")


@functools.partial(
    pl.kernel,
    out_type=jax.ShapeDtypeStruct((TOT, DIM), jnp.float32),
    mesh=_mesh,
    scratch_types=[
        pltpu.VMEM((POS_W,), jnp.int32),
        pltpu.VMEM_SHARED((SEQ, GW), jnp.float32),
        pltpu.VMEM_SHARED((HALF, GW), jnp.float32),
        [pltpu.VMEM((CPOS, GW), jnp.float32)] * 3,
        pltpu.SemaphoreType.DMA,
        pltpu.SemaphoreType.DMA,
        [pltpu.SemaphoreType.DMA] * 3,
        [pltpu.SemaphoreType.DMA] * 3,
    ],
)
def _gather_sc(ids_hbm, table_hbm, out_hbm, idx_v, slab, half, tbufs,
               ssem, hsem, gsems, psems):
    cid = lax.axis_index("c")
    sid = lax.axis_index("s")
    pos0 = sid * POS_W

    pltpu.sync_copy(ids_hbm.at[sid], idx_v)

    def col0(g):
        return (cid * NGRP + g) * GW

    def stage_lower(g):
        r0 = sid * SPC
        src = table_hbm.at[pl.ds(r0, SPC), pl.ds(col0(g), GW)]
        pltpu.async_copy(src, slab.at[pl.ds(r0, SPC)], ssem)

    def swait():
        src = table_hbm.at[pl.ds(0, SPC), pl.ds(0, GW)]
        pltpu.make_async_copy(src, slab.at[pl.ds(0, SPC)], ssem).wait()

    def prefetch(g):
        r0 = sid * SPC
        src = table_hbm.at[pl.ds(HALF + r0, SPC), pl.ds(col0(g), GW)]
        pltpu.async_copy(src, half.at[pl.ds(r0, SPC)], hsem)

    def hwait():
        src = table_hbm.at[pl.ds(0, SPC), pl.ds(0, GW)]
        pltpu.make_async_copy(src, half.at[pl.ds(0, SPC)], hsem).wait()

    def relay():
        ntrip = SPC // CPOS
        for r in range(ntrip):
            tb = r % 3
            r0 = sid * SPC + r * CPOS
            if r >= 3:
                pltpu.make_async_copy(
                    tbufs[tb], slab.at[pl.ds(HALF, CPOS)], psems[tb]
                ).wait()
            pltpu.async_copy(half.at[pl.ds(r0, CPOS)], tbufs[tb], gsems[tb])
            pltpu.make_async_copy(
                half.at[pl.ds(r0, CPOS)], tbufs[tb], gsems[tb]
            ).wait()
            pltpu.async_copy(
                tbufs[tb], slab.at[pl.ds(HALF + r0, CPOS)], psems[tb]
            )
        for r in range(ntrip - 3, ntrip):
            tb = r % 3
            pltpu.make_async_copy(
                tbufs[tb], slab.at[pl.ds(HALF, CPOS)], psems[tb]
            ).wait()

    def gather(ch, tb):
        idx = idx_v.at[pl.ds(ch * CPOS, CPOS)]
        pltpu.async_copy(slab.at[idx], tbufs[tb], gsems[tb])

    def gwait(tb):
        idx = idx_v.at[pl.ds(0, CPOS)]
        pltpu.make_async_copy(slab.at[idx], tbufs[tb], gsems[tb]).wait()

    def put(g, ch, tb):
        dst = out_hbm.at[pl.ds(pos0 + ch * CPOS, CPOS), pl.ds(col0(g), GW)]
        pltpu.async_copy(tbufs[tb], dst, psems[tb])

    def pwait(tb):
        dst = out_hbm.at[pl.ds(0, CPOS), pl.ds(0, GW)]
        pltpu.make_async_copy(tbufs[tb], dst, psems[tb]).wait()

    def process(g):
        gather(0, 0)
        gather(1, 1)
        gather(2, 2)
        for ch in range(NCH):
            tb = ch % 3
            gwait(tb)
            put(g, ch, tb)
            if ch == 2:
                @pl.when(g < NGRP - 1)
                def _():
                    prefetch(g + 1)
            if ch + 3 < NCH:
                pwait(tb)
                gather(ch + 3, tb)

    stage_lower(0)
    prefetch(0)
    hwait()
    relay()

    def body(g, carry):
        swait()
        plsc.subcore_barrier()
        process(g)
        plsc.subcore_barrier()

        @pl.when(g < NGRP - 1)
        def _():
            stage_lower(g + 1)

        pwait(0)
        pwait(1)
        pwait(2)

        @pl.when(g < NGRP - 1)
        def _():
            hwait()
            relay()
        return carry

    lax.fori_loop(0, NGRP, body, 0)


def kernel(position_ids, table):
    ids = position_ids.reshape(NS, POS_W).astype(jnp.int32)
    out = _gather_sc(ids, table)
    return out.reshape(position_ids.shape[0], position_ids.shape[1], DIM)
